# baseline (speedup 1.0000x reference)
import functools

import jax
import jax.numpy as jnp
from jax import lax
from jax.experimental import pallas as pl
from jax.experimental.pallas import tpu as pltpu
from jax.experimental.pallas import tpu_sc as plsc

B = 16384
D = 16

_info = plsc.get_sparse_core_info()
_NC, _NS = _info.num_cores, _info.num_subcores
_NW = _NC * _NS
_BPW = B // _NW


def _sc_gather_body(x0_hbm, x1_hbm, x2_hbm, x3_hbm,
                    t0_hbm, t1_hbm, t2_hbm, t3_hbm,
                    o0_hbm, o1_hbm, o2_hbm, o3_hbm,
                    i0_v, i1_v, i2_v, i3_v,
                    r0_v, r1_v, r2_v, r3_v,
                    s0, s1, s2, s3):
  wid = lax.axis_index("s") * _NC + lax.axis_index("c")
  base = wid * _BPW
  pltpu.sync_copy(x0_hbm.at[pl.ds(base, _BPW)], i0_v)
  pltpu.sync_copy(x1_hbm.at[pl.ds(base, _BPW)], i1_v)
  pltpu.sync_copy(x2_hbm.at[pl.ds(base, _BPW)], i2_v)
  pltpu.sync_copy(x3_hbm.at[pl.ds(base, _BPW)], i3_v)
  c0 = pltpu.async_copy(t0_hbm.at[i0_v], r0_v, s0)
  c1 = pltpu.async_copy(t1_hbm.at[i1_v], r1_v, s1)
  c2 = pltpu.async_copy(t2_hbm.at[i2_v], r2_v, s2)
  c3 = pltpu.async_copy(t3_hbm.at[i3_v], r3_v, s3)
  c0.wait()
  pltpu.sync_copy(r0_v, o0_hbm.at[pl.ds(base, _BPW)])
  c1.wait()
  pltpu.sync_copy(r1_v, o1_hbm.at[pl.ds(base, _BPW)])
  c2.wait()
  pltpu.sync_copy(r2_v, o2_hbm.at[pl.ds(base, _BPW)])
  c3.wait()
  pltpu.sync_copy(r3_v, o3_hbm.at[pl.ds(base, _BPW)])


_sc_gather = functools.partial(
    pl.kernel,
    mesh=plsc.VectorSubcoreMesh(core_axis_name="c", subcore_axis_name="s"),
    compiler_params=pltpu.CompilerParams(use_tc_tiling_on_sc=False),
    out_type=[jax.ShapeDtypeStruct((B, D), jnp.float32)] * 4,
    scratch_types=(
        [pltpu.VMEM((_BPW,), jnp.int32)] * 4
        + [pltpu.VMEM((_BPW, D), jnp.float32)] * 4
        + [pltpu.SemaphoreType.DMA] * 4
    ),
)(_sc_gather_body)


_ROWS = B // 8
_BLK = 256


def _tc_head_body(gu, gi, mu, mi, w1t, w1b, b1t, w2bd, b2t, wgf, w3f, cb,
                  out):
  f32 = jnp.float32
  g = gu[...] * gi[...]
  h1 = jnp.dot(mu[...], w1t[...], preferred_element_type=f32)
  h1 = h1 + jnp.dot(mi[...], w1b[...], preferred_element_type=f32)
  h1 = jnp.maximum(h1 + b1t[...], 0.0)
  h2 = jnp.maximum(
      jnp.dot(h1, w2bd[...], preferred_element_type=f32) + b2t[...], 0.0)
  z = jnp.dot(g, wgf[...], preferred_element_type=f32)
  z = z + jnp.dot(h2, w3f[...], preferred_element_type=f32) + cb[...]
  out[...] = jax.nn.sigmoid(z)


def _tc_head(gu, gi, mu, mi, w1t, w1b, b1t, w2bd, b2t, wgf, w3f, cb):
  n_steps = _ROWS // _BLK
  data_spec = pl.BlockSpec((_BLK, 128), lambda i: (i, 0))
  full = lambda a: pl.BlockSpec(a.shape, lambda i: (0,) * a.ndim)
  return pl.pallas_call(
      _tc_head_body,
      grid=(n_steps,),
      in_specs=[data_spec] * 4 + [full(w1t), full(w1b), full(b1t),
                                  full(w2bd), full(b2t), full(wgf),
                                  full(w3f), full(cb)],
      out_specs=pl.BlockSpec((_BLK, 8), lambda i: (i, 0)),
      out_shape=jax.ShapeDtypeStruct((_ROWS, 8), jnp.float32),
  )(gu, gi, mu, mi, w1t, w1b, b1t, w2bd, b2t, wgf, w3f, cb)


def kernel(x0, x1, x2, x3, gmf_user_emb, gmf_item_emb, gmf_w, gmf_b,
           mlp_user_emb, mlp_item_emb, w1, b1, w2, b2, w3, b3, cls_w, cls_b):
  i0 = x0.reshape(B).astype(jnp.int32)
  i1 = x1.reshape(B).astype(jnp.int32)
  i2 = x2.reshape(B).astype(jnp.int32)
  i3 = x3.reshape(B).astype(jnp.int32)

  gu, gi, mu, mi = _sc_gather(i0, i1, i2, i3, gmf_user_emb, gmf_item_emb,
                              mlp_user_emb, mlp_item_emb)

  gu = gu.reshape(_ROWS, 128)
  gi = gi.reshape(_ROWS, 128)
  mu = mu.reshape(_ROWS, 128)
  mi = mi.reshape(_ROWS, 128)

  eye8 = jnp.eye(8, dtype=jnp.float32)
  w1t = jnp.kron(eye8, w1[:D, :])
  w1b = jnp.kron(eye8, w1[D:, :])
  b1t = jnp.tile(b1, 8).reshape(1, 256)
  w2bd = jnp.kron(eye8, w2)
  b2t = jnp.tile(b2, 8).reshape(1, 128)
  cw0 = cls_w[0, 0]
  cw1 = cls_w[1, 0]
  wgf = jnp.kron(eye8, gmf_w * cw0)
  w3f = jnp.kron(eye8, w3 * cw1)
  cb = jnp.full((1, 8), gmf_b[0] * cw0 + b3[0] * cw1 + cls_b[0],
                dtype=jnp.float32)

  out = _tc_head(gu, gi, mu, mi, w1t, w1b, b1t, w2bd, b2t, wgf, w3f, cb)
  return out.reshape(B, 1)
